# baseline (speedup 1.0000x reference)
import functools

import numpy as np
import jax
import jax.numpy as jnp
from jax import lax
from jax.experimental import pallas as pl
from jax.experimental.pallas import tpu as pltpu
from jax.experimental.pallas import tpu_sc as plsc

D_MODEL = 64
MAX_LEN = 2048
SEQ_LEN = 2048
BATCH = 16

NUM_WORKERS = 32
ROWS = SEQ_LEN * BATCH
RPW = ROWS // NUM_WORKERS
CHUNK = 128
NCHUNK = RPW // CHUNK
POS_PER_CHUNK = CHUNK // BATCH
POS_PER_WORKER = RPW // BATCH
NLANE = D_MODEL // 16
NBUF = 4


def _make_pe_table(max_len, d_model):
    position = np.arange(0, max_len, dtype=np.float32)[:, None]
    div_term = np.exp(
        np.arange(0, d_model, 2).astype(np.float32) * (-np.log(10000.0) / d_model)
    )
    pe = np.zeros((max_len, d_model), dtype=np.float32)
    pe[:, 0::2] = np.sin(position * div_term)
    pe[:, 1::2] = np.cos(position * div_term)
    return pe


_PE_NP = np.zeros((SEQ_LEN, 128), dtype=np.float32)
_PE_NP[:, :D_MODEL] = _make_pe_table(MAX_LEN, D_MODEL)[:SEQ_LEN]

_mesh = plsc.VectorSubcoreMesh(core_axis_name="c", subcore_axis_name="s")


@functools.partial(
    pl.kernel,
    mesh=_mesh,
    compiler_params=pltpu.CompilerParams(use_tc_tiling_on_sc=False),
    out_type=jax.ShapeDtypeStruct((SEQ_LEN, BATCH, D_MODEL), jnp.float32),
    scratch_types=[
        pltpu.VMEM((POS_PER_WORKER, 128), jnp.int32),
        pltpu.VMEM((POS_PER_WORKER, 128), jnp.float32),
        pltpu.VMEM((NBUF, POS_PER_CHUNK, BATCH, D_MODEL), jnp.float32),
        pltpu.SemaphoreType.DMA((NBUF,)),
        pltpu.SemaphoreType.DMA((NBUF,)),
    ],
)
def _embed_sc(x_hbm, w_hbm, pe_hbm, out_hbm, idxa, pe_v, bufs, gsem, osem):
    wid = lax.axis_index("s") * 2 + lax.axis_index("c")
    base_pos = wid * POS_PER_WORKER

    pltpu.sync_copy(x_hbm.at[pl.ds(base_pos, POS_PER_WORKER)], idxa)
    pltpu.sync_copy(pe_hbm.at[pl.ds(base_pos, POS_PER_WORKER)], pe_v)

    def fire_gathers(j):
        b = j % NBUF
        descs = []
        for p in range(POS_PER_CHUNK):
            iv = idxa[j * POS_PER_CHUNK + p, pl.ds(0, BATCH)]
            descs.append(
                pltpu.async_copy(w_hbm.at[iv], bufs.at[b, p], gsem.at[b])
            )
        return descs

    gathers = [fire_gathers(0), fire_gathers(1)]
    writebacks = [None] * NCHUNK
    for j in range(NCHUNK):
        b = j % NBUF
        if j - NBUF >= 0:
            writebacks[j - NBUF].wait()
        if j + 2 < NCHUNK:
            gathers.append(fire_gathers(j + 2))
        for d in gathers[j]:
            d.wait()

        def body(p, _):
            prow = j * POS_PER_CHUNK + p
            pes = [pe_v[prow, pl.ds(c * 16, 16)] for c in range(NLANE)]
            for t in range(BATCH):
                for c in range(NLANE):
                    sl = pl.ds(c * 16, 16)
                    bufs[b, p, t, sl] = bufs[b, p, t, sl] + pes[c]
            return 0

        lax.fori_loop(0, POS_PER_CHUNK, body, 0)
        writebacks[j] = pltpu.async_copy(
            bufs.at[b],
            out_hbm.at[pl.ds(base_pos + j * POS_PER_CHUNK, POS_PER_CHUNK)],
            osem.at[b],
        )
    for j in range(NCHUNK - NBUF, NCHUNK):
        writebacks[j].wait()


def kernel(x, W):
    xp = jnp.pad(x, ((0, 0), (0, 128 - BATCH)))
    return _embed_sc(xp, W, jnp.asarray(_PE_NP))


# speedup vs baseline: 1.3878x; 1.3878x over previous
import functools

import numpy as np
import jax
import jax.numpy as jnp
from jax import lax
from jax.experimental import pallas as pl
from jax.experimental.pallas import tpu as pltpu
from jax.experimental.pallas import tpu_sc as plsc

D_MODEL = 64
MAX_LEN = 2048
SEQ_LEN = 2048
BATCH = 16

NUM_WORKERS = 32
ROWS = SEQ_LEN * BATCH
RPW = ROWS // NUM_WORKERS
CHUNK = 128
NCHUNK = RPW // CHUNK
POS_PER_CHUNK = CHUNK // BATCH
POS_PER_WORKER = RPW // BATCH
NLANE = D_MODEL // 16
NBUF = 4


def _make_pe_table(max_len, d_model):
    position = np.arange(0, max_len, dtype=np.float32)[:, None]
    div_term = np.exp(
        np.arange(0, d_model, 2).astype(np.float32) * (-np.log(10000.0) / d_model)
    )
    pe = np.zeros((max_len, d_model), dtype=np.float32)
    pe[:, 0::2] = np.sin(position * div_term)
    pe[:, 1::2] = np.cos(position * div_term)
    return pe


_PE_NP = np.zeros((SEQ_LEN, 128), dtype=np.float32)
_PE_NP[:, :D_MODEL] = _make_pe_table(MAX_LEN, D_MODEL)[:SEQ_LEN]

_mesh = plsc.VectorSubcoreMesh(core_axis_name="c", subcore_axis_name="s")


@functools.partial(
    pl.kernel,
    mesh=_mesh,
    out_type=jax.ShapeDtypeStruct((SEQ_LEN, BATCH, D_MODEL), jnp.float32),
    scratch_types=[
        pltpu.VMEM((POS_PER_WORKER, BATCH), jnp.int32),
        pltpu.VMEM((POS_PER_WORKER, 128), jnp.float32),
        pltpu.VMEM((NBUF, POS_PER_CHUNK, BATCH, D_MODEL), jnp.float32),
        pltpu.VMEM((CHUNK, D_MODEL), jnp.float32),
        pltpu.SemaphoreType.DMA((NBUF,)),
        pltpu.SemaphoreType.DMA((NBUF,)),
    ],
)
def _embed_sc(x_hbm, w_hbm, pe_hbm, out_hbm, sidx, pe_v, bufs, drain, gsem, osem):
    wid = lax.axis_index("s") * 2 + lax.axis_index("c")
    base_pos = wid * POS_PER_WORKER

    pltpu.sync_copy(x_hbm.at[pl.ds(base_pos, POS_PER_WORKER)], sidx)
    pltpu.sync_copy(pe_hbm.at[pl.ds(base_pos, POS_PER_WORKER)], pe_v)

    def fire_gathers(j):
        b = j % NBUF

        def fire_pos(p, _):
            iv = sidx[j * POS_PER_CHUNK + p, :]
            for t in range(BATCH):
                pltpu.async_copy(
                    w_hbm.at[pl.ds(iv[t], 1)],
                    bufs.at[b, p, pl.ds(t, 1)],
                    gsem.at[b],
                )
            return 0

        lax.fori_loop(0, POS_PER_CHUNK, fire_pos, 0)

    def drain_gathers(j):
        pltpu.make_async_copy(
            w_hbm.at[pl.ds(0, CHUNK)], drain, gsem.at[j % NBUF]
        ).wait()

    fire_gathers(0)
    fire_gathers(1)
    writebacks = [None] * NCHUNK
    for j in range(NCHUNK):
        b = j % NBUF
        if j - NBUF >= 0:
            writebacks[j - NBUF].wait()
        if j + 2 < NCHUNK:
            fire_gathers(j + 2)
        drain_gathers(j)

        def body(p, _):
            prow = j * POS_PER_CHUNK + p
            pes = [pe_v[prow, pl.ds(c * 16, 16)] for c in range(NLANE)]
            for t in range(BATCH):
                for c in range(NLANE):
                    sl = pl.ds(c * 16, 16)
                    bufs[b, p, t, sl] = bufs[b, p, t, sl] + pes[c]
            return 0

        lax.fori_loop(0, POS_PER_CHUNK, body, 0)
        writebacks[j] = pltpu.async_copy(
            bufs.at[b],
            out_hbm.at[pl.ds(base_pos + j * POS_PER_CHUNK, POS_PER_CHUNK)],
            osem.at[b],
        )
    for j in range(NCHUNK - NBUF, NCHUNK):
        writebacks[j].wait()


def kernel(x, W):
    return _embed_sc(x, W, jnp.asarray(_PE_NP))


# speedup vs baseline: 1.3919x; 1.0030x over previous
import functools

import numpy as np
import jax
import jax.numpy as jnp
from jax import lax
from jax.experimental import pallas as pl
from jax.experimental.pallas import tpu as pltpu
from jax.experimental.pallas import tpu_sc as plsc

D_MODEL = 64
MAX_LEN = 2048
SEQ_LEN = 2048
BATCH = 16

NUM_WORKERS = 32
ROWS = SEQ_LEN * BATCH
RPW = ROWS // NUM_WORKERS
CHUNK = 128
NCHUNK = RPW // CHUNK
POS_PER_CHUNK = CHUNK // BATCH
POS_PER_WORKER = RPW // BATCH
NLANE = D_MODEL // 16
NBUF = 4


def _make_pe_table(max_len, d_model):
    position = np.arange(0, max_len, dtype=np.float32)[:, None]
    div_term = np.exp(
        np.arange(0, d_model, 2).astype(np.float32) * (-np.log(10000.0) / d_model)
    )
    pe = np.zeros((max_len, d_model), dtype=np.float32)
    pe[:, 0::2] = np.sin(position * div_term)
    pe[:, 1::2] = np.cos(position * div_term)
    return pe


_PE_NP = np.zeros((SEQ_LEN, 128), dtype=np.float32)
_PE_NP[:, :D_MODEL] = _make_pe_table(MAX_LEN, D_MODEL)[:SEQ_LEN]

_mesh = plsc.VectorSubcoreMesh(core_axis_name="c", subcore_axis_name="s")


@functools.partial(
    pl.kernel,
    mesh=_mesh,
    out_type=jax.ShapeDtypeStruct((SEQ_LEN, BATCH, D_MODEL), jnp.float32),
    scratch_types=[
        pltpu.VMEM((POS_PER_WORKER, BATCH), jnp.int32),
        pltpu.VMEM((POS_PER_WORKER, 128), jnp.float32),
        pltpu.VMEM((NBUF, POS_PER_CHUNK, BATCH, D_MODEL), jnp.float32),
        pltpu.VMEM((CHUNK, D_MODEL), jnp.float32),
        pltpu.SemaphoreType.DMA((NBUF,)),
        pltpu.SemaphoreType.DMA((NBUF,)),
    ],
)
def _embed_sc(x_hbm, w_hbm, pe_hbm, out_hbm, sidx, pe_v, bufs, drain, gsem, osem):
    wid = lax.axis_index("s") * 2 + lax.axis_index("c")
    base_pos = wid * POS_PER_WORKER

    pltpu.sync_copy(x_hbm.at[pl.ds(base_pos, POS_PER_WORKER)], sidx)
    pltpu.sync_copy(pe_hbm.at[pl.ds(base_pos, POS_PER_WORKER)], pe_v)

    def fire_gathers(j):
        b = j % NBUF

        def fire_pos(p, _):
            iv = sidx[j * POS_PER_CHUNK + p, :]
            for t in range(BATCH):
                pltpu.async_copy(
                    w_hbm.at[pl.ds(iv[t], 1)],
                    bufs.at[b, p, pl.ds(t, 1)],
                    gsem.at[b],
                )
            return 0

        lax.fori_loop(0, POS_PER_CHUNK, fire_pos, 0)

    def drain_gathers(j):
        pltpu.make_async_copy(
            w_hbm.at[pl.ds(0, CHUNK)], drain, gsem.at[j % NBUF]
        ).wait()

    fire_gathers(0)
    fire_gathers(1)
    writebacks = [None] * NCHUNK
    for j in range(NCHUNK):
        b = j % NBUF
        if j + 2 < NCHUNK:
            if j - 2 >= 0:
                writebacks[j - 2].wait()
            fire_gathers(j + 2)
        drain_gathers(j)

        def body(p, _):
            prow = j * POS_PER_CHUNK + p
            pes = [pe_v[prow, pl.ds(c * 16, 16)] for c in range(NLANE)]
            for t in range(BATCH):
                for c in range(NLANE):
                    sl = pl.ds(c * 16, 16)
                    bufs[b, p, t, sl] = bufs[b, p, t, sl] + pes[c]
            return 0

        lax.fori_loop(0, POS_PER_CHUNK, body, 0)
        writebacks[j] = pltpu.async_copy(
            bufs.at[b],
            out_hbm.at[pl.ds(base_pos + j * POS_PER_CHUNK, POS_PER_CHUNK)],
            osem.at[b],
        )
    for j in range(NCHUNK - 4, NCHUNK):
        writebacks[j].wait()


def kernel(x, W):
    return _embed_sc(x, W, jnp.asarray(_PE_NP))
